# Initial kernel scaffold; baseline (speedup 1.0000x reference)
import jax
import jax.numpy as jnp
from jax.experimental import pallas as pl


def kernel(edge_index, edge_type, weight1, root1, bias1, weight2, root2, bias2, lin_w, lin_b):
    raise NotImplementedError("write your pallas kernel here")


# speedup vs baseline: 2.3488x; 2.3488x over previous
import jax
import jax.numpy as jnp
from jax import lax
from jax.experimental import pallas as pl
from jax.experimental.pallas import tpu as pltpu
from jax.experimental.pallas import tpu_sc as plsc

N = 10000
E = 320000
R = 16
H = 128
OUT = 256
RN = R * N

NC = 2
NS = 16
NW = NC * NS

CH = 128
NROWS = 2560
EP = NROWS * CH
PAD = EP - E
RPW = NROWS // NW
RPT = NROWS // NS
LCH = 16
GP = CH // 16

RNP = RN + 256
SPT = RNP // NS
NP = 10240
NPT = NP // NS
ZR = 128

_f32 = jnp.float32
_i32 = jnp.int32


def _count_body(src_hbm, dst_hbm, rel_hbm,
                w1idx_hbm, comb_hbm, inv_hbm,
                srcb, dstb, relb, widxb, combb, onesb, fbuf, cnt_sh):
    c = lax.axis_index("c")
    s = lax.axis_index("s")

    for i in range(GP):
        onesb[pl.ds(i * 16, 16)] = jnp.ones((16,), _f32)

    def zstep(i, _):
        fbuf[pl.ds(i * 16, 16)] = jnp.zeros((16,), _f32)
        return 0
    lax.fori_loop(0, SPT // 16, zstep, 0)

    @pl.when(c == 0)
    def _zero():
        pltpu.sync_copy(fbuf, cnt_sh.at[pl.ds(s * SPT, SPT)])

    plsc.subcore_barrier()

    def chunk(ci, _):
        r0 = s * RPT + ci * LCH
        pltpu.sync_copy(src_hbm.at[pl.ds(r0, LCH)], srcb)
        pltpu.sync_copy(dst_hbm.at[pl.ds(r0, LCH)], dstb)
        pltpu.sync_copy(rel_hbm.at[pl.ds(r0, LCH)], relb)

        def vec(i, _):
            rr = i // GP
            cc = (i % GP) * 16
            sv = srcb[rr, pl.ds(cc, 16)]
            dv = dstb[rr, pl.ds(cc, 16)]
            rv = relb[rr, pl.ds(cc, 16)]
            widxb[rr, pl.ds(cc, 16)] = rv * N + sv
            combb[rr, pl.ds(cc, 16)] = rv * N + dv
            return 0
        lax.fori_loop(0, LCH * GP, vec, 0)

        @pl.when(c == 1)
        def _store_idx():
            pltpu.sync_copy(widxb, w1idx_hbm.at[pl.ds(r0, LCH)])
            pltpu.sync_copy(combb, comb_hbm.at[pl.ds(r0, LCH)])

        @pl.when(c == 0)
        def _hist():
            def hist(rr, _):
                pltpu.sync_copy(onesb, cnt_sh.at[combb.at[rr]], add=True)
                return 0
            lax.fori_loop(0, LCH, hist, 0)
        return 0
    lax.fori_loop(0, RPT // LCH, chunk, 0)

    plsc.subcore_barrier()

    @pl.when(c == 0)
    def _inv():
        pltpu.sync_copy(cnt_sh.at[pl.ds(s * SPT, SPT)], fbuf)

        def invstep(i, _):
            v = fbuf[pl.ds(i * 16, 16)]
            fbuf[pl.ds(i * 16, 16)] = 1.0 / jnp.maximum(v, 1.0)
            return 0
        lax.fori_loop(0, SPT // 16, invstep, 0)
        pltpu.sync_copy(fbuf, inv_hbm.at[pl.ds(s * SPT, SPT)])


_count_call = pl.kernel(
    _count_body,
    out_type=(
        jax.ShapeDtypeStruct((NROWS, CH), _i32),
        jax.ShapeDtypeStruct((NROWS, CH), _i32),
        jax.ShapeDtypeStruct((RNP,), _f32),
    ),
    mesh=plsc.VectorSubcoreMesh(core_axis_name="c", subcore_axis_name="s"),
    scratch_types=[
        pltpu.VMEM((LCH, CH), _i32),
        pltpu.VMEM((LCH, CH), _i32),
        pltpu.VMEM((LCH, CH), _i32),
        pltpu.VMEM((LCH, CH), _i32),
        pltpu.VMEM((LCH, CH), _i32),
        pltpu.VMEM((CH,), _f32),
        pltpu.VMEM((SPT,), _f32),
        pltpu.VMEM_SHARED((RNP,), _f32),
    ],
)


def _scale_body(comb_hbm, inv_hbm, scale_hbm, combb, sbuf, sem):
    c = lax.axis_index("c")
    s = lax.axis_index("s")
    r0 = (s * NC + c) * RPW
    pltpu.sync_copy(comb_hbm.at[pl.ds(r0, RPW)], combb)

    def step(j, _):
        pltpu.async_copy(inv_hbm.at[combb.at[j]], sbuf.at[j], sem).wait()
        return 0
    lax.fori_loop(0, RPW, step, 0)
    pltpu.sync_copy(sbuf, scale_hbm.at[pl.ds(r0, RPW)])


_scale_call = pl.kernel(
    _scale_body,
    out_type=jax.ShapeDtypeStruct((NROWS, CH), _f32),
    mesh=plsc.VectorSubcoreMesh(core_axis_name="c", subcore_axis_name="s"),
    scratch_types=[
        pltpu.VMEM((RPW, CH), _i32),
        pltpu.VMEM((RPW, CH), _f32),
        pltpu.SemaphoreType.DMA,
    ],
)


def _agg_body(table_hbm, idx_hbm, scale_hbm, dst_hbm, out_hbm,
              idxb, scaleb, dstb, rowsb, agg_sh, sem):
    c = lax.axis_index("c")
    s = lax.axis_index("s")
    r0 = (s * NC + c) * RPW

    def zstep(i, _):
        for k in range(GP):
            rowsb[i, pl.ds(k * 16, 16)] = jnp.zeros((16,), _f32)
        return 0
    lax.fori_loop(0, CH, zstep, 0)
    for k in range(NPT // ZR):
        pltpu.sync_copy(rowsb, agg_sh.at[pl.ds(s * NPT + k * ZR, ZR), :])
    plsc.subcore_barrier()

    pltpu.sync_copy(idx_hbm.at[pl.ds(r0, RPW)], idxb)
    pltpu.sync_copy(scale_hbm.at[pl.ds(r0, RPW)], scaleb)
    pltpu.sync_copy(dst_hbm.at[pl.ds(r0, RPW)], dstb)

    def step(j, _):
        pltpu.async_copy(table_hbm.at[idxb.at[j]], rowsb, sem).wait()

        def edge_group(g, _):
            scv = scaleb[j, pl.ds(g * 16, 16)]
            for l in range(16):
                sc = scv[l]
                i = g * 16 + l
                for k in range(H // 16):
                    v = rowsb[i, pl.ds(k * 16, 16)]
                    rowsb[i, pl.ds(k * 16, 16)] = v * sc
            return 0
        lax.fori_loop(0, GP, edge_group, 0)

        pltpu.sync_copy(rowsb, agg_sh.at[dstb.at[j]], add=True)
        return 0
    lax.fori_loop(0, RPW, step, 0)

    plsc.subcore_barrier()
    for k in range(NPT // ZR):
        b0 = s * NPT + k * ZR
        pltpu.sync_copy(agg_sh.at[pl.ds(b0, ZR), :], rowsb)
        pltpu.sync_copy(rowsb, out_hbm.at[c, pl.ds(b0, ZR), :])


_agg_call = pl.kernel(
    _agg_body,
    out_type=jax.ShapeDtypeStruct((NC, NP, H), _f32),
    mesh=plsc.VectorSubcoreMesh(core_axis_name="c", subcore_axis_name="s"),
    scratch_types=[
        pltpu.VMEM((RPW, CH), _i32),
        pltpu.VMEM((RPW, CH), _f32),
        pltpu.VMEM((RPW, CH), _i32),
        pltpu.VMEM((CH, H), _f32),
        pltpu.VMEM_SHARED((NP, H), _f32),
        pltpu.SemaphoreType.DMA,
    ],
)


BN = 1000


def _xr_body(a0, a1, r1, b1, w2, x_out, xr_out):
    xb = a0[...] + a1[...] + r1[...] + b1[...][None, :]
    xb = jnp.where(xb > 0, xb, jnp.exp(xb) - 1.0)
    x_out[...] = xb
    for r in range(R):
        xr_out[r] = jnp.dot(xb, w2[r], preferred_element_type=_f32)


_xr_call = pl.pallas_call(
    _xr_body,
    grid=(N // BN,),
    in_specs=[
        pl.BlockSpec((BN, H), lambda i: (i, 0)),
        pl.BlockSpec((BN, H), lambda i: (i, 0)),
        pl.BlockSpec((BN, H), lambda i: (i, 0)),
        pl.BlockSpec((H,), lambda i: (0,)),
        pl.BlockSpec((R, H, H), lambda i: (0, 0, 0)),
    ],
    out_specs=[
        pl.BlockSpec((BN, H), lambda i: (i, 0)),
        pl.BlockSpec((R, BN, H), lambda i: (0, i, 0)),
    ],
    out_shape=[
        jax.ShapeDtypeStruct((N, H), _f32),
        jax.ShapeDtypeStruct((R, N, H), _f32),
    ],
)


def _fin_body(a0, a1, x, r2, b2, lw, lb, o):
    y = a0[...] + a1[...] + jnp.dot(x[...], r2[...], preferred_element_type=_f32)
    y = y + b2[...][None, :]
    y = jnp.where(y > 0, y, jnp.exp(y) - 1.0)
    o[...] = jnp.dot(y, lw[...], preferred_element_type=_f32) + lb[...][None, :]


_fin_call = pl.pallas_call(
    _fin_body,
    grid=(N // BN,),
    in_specs=[
        pl.BlockSpec((BN, H), lambda i: (i, 0)),
        pl.BlockSpec((BN, H), lambda i: (i, 0)),
        pl.BlockSpec((BN, H), lambda i: (i, 0)),
        pl.BlockSpec((H, H), lambda i: (0, 0)),
        pl.BlockSpec((H,), lambda i: (0,)),
        pl.BlockSpec((H, OUT), lambda i: (0, 0)),
        pl.BlockSpec((OUT,), lambda i: (0,)),
    ],
    out_specs=pl.BlockSpec((BN, OUT), lambda i: (i, 0)),
    out_shape=jax.ShapeDtypeStruct((N, OUT), _f32),
)


def kernel(edge_index, edge_type, weight1, root1, bias1, weight2, root2,
           bias2, lin_w, lin_b):
    src2 = jnp.concatenate(
        [edge_index[0], jnp.zeros((PAD,), _i32)]).reshape(NROWS, CH)
    dst2 = jnp.concatenate(
        [edge_index[1], jnp.full((PAD,), N, _i32)]).reshape(NROWS, CH)
    rel2 = jnp.concatenate(
        [edge_type, jnp.full((PAD,), R - 1, _i32)]).reshape(NROWS, CH)
    w1idx2, comb2, inv = _count_call(src2, dst2, rel2)
    scale2 = _scale_call(comb2, inv)

    agg1 = _agg_call(weight1.reshape(RN, H), w1idx2, scale2, dst2)
    x, xr = _xr_call(agg1[0, :N], agg1[1, :N], root1, bias1, weight2)

    agg2 = _agg_call(xr.reshape(RN, H), w1idx2, scale2, dst2)
    out = _fin_call(agg2[0, :N], agg2[1, :N], x, root2, bias2, lin_w, lin_b)
    return out


# speedup vs baseline: 3.1842x; 1.3557x over previous
import jax
import jax.numpy as jnp
from jax import lax
from jax.experimental import pallas as pl
from jax.experimental.pallas import tpu as pltpu
from jax.experimental.pallas import tpu_sc as plsc

N = 10000
E = 320000
R = 16
H = 128
OUT = 256
RN = R * N

NC = 2
NS = 16
NW = NC * NS

CH = 128
NROWS = 2560
EP = NROWS * CH
PAD = EP - E
RPW = NROWS // NW
RPT = NROWS // NS
LCH = 16
GP = CH // 16

RNP = RN + 256
SPT = RNP // NS
NP = 10240
NPT = NP // NS
ZR = 128
A0 = 120

_f32 = jnp.float32
_i32 = jnp.int32


def _count_body(src_hbm, dst_hbm, rel_hbm,
                w1idx_hbm, comb_hbm, inv_hbm,
                srcb, dstb, relb, widxb, combb, onesb, fbuf, cnt_sh):
    c = lax.axis_index("c")
    s = lax.axis_index("s")

    for i in range(GP):
        onesb[pl.ds(i * 16, 16)] = jnp.ones((16,), _f32)

    def zstep(i, _):
        fbuf[pl.ds(i * 16, 16)] = jnp.zeros((16,), _f32)
        return 0
    lax.fori_loop(0, SPT // 16, zstep, 0)

    @pl.when(c == 0)
    def _zero():
        pltpu.sync_copy(fbuf, cnt_sh.at[pl.ds(s * SPT, SPT)])

    plsc.subcore_barrier()

    def chunk(ci, _):
        r0 = s * RPT + ci * LCH
        pltpu.sync_copy(src_hbm.at[pl.ds(r0, LCH)], srcb)
        pltpu.sync_copy(dst_hbm.at[pl.ds(r0, LCH)], dstb)
        pltpu.sync_copy(rel_hbm.at[pl.ds(r0, LCH)], relb)

        def vec(i, _):
            rr = i // GP
            cc = (i % GP) * 16
            sv = srcb[rr, pl.ds(cc, 16)]
            dv = dstb[rr, pl.ds(cc, 16)]
            rv = relb[rr, pl.ds(cc, 16)]
            widxb[rr, pl.ds(cc, 16)] = rv * N + sv
            combb[rr, pl.ds(cc, 16)] = rv * N + dv
            return 0
        lax.fori_loop(0, LCH * GP, vec, 0)

        @pl.when(c == 1)
        def _store_idx():
            pltpu.sync_copy(widxb, w1idx_hbm.at[pl.ds(r0, LCH)])
            pltpu.sync_copy(combb, comb_hbm.at[pl.ds(r0, LCH)])

        @pl.when(c == 0)
        def _hist():
            def hist(rr, _):
                pltpu.sync_copy(onesb, cnt_sh.at[combb.at[rr]], add=True)
                return 0
            lax.fori_loop(0, LCH, hist, 0)
        return 0
    lax.fori_loop(0, RPT // LCH, chunk, 0)

    plsc.subcore_barrier()

    @pl.when(c == 0)
    def _inv():
        pltpu.sync_copy(cnt_sh.at[pl.ds(s * SPT, SPT)], fbuf)

        def invstep(i, _):
            v = fbuf[pl.ds(i * 16, 16)]
            fbuf[pl.ds(i * 16, 16)] = 1.0 / jnp.maximum(v, 1.0)
            return 0
        lax.fori_loop(0, SPT // 16, invstep, 0)
        pltpu.sync_copy(fbuf, inv_hbm.at[pl.ds(s * SPT, SPT)])


_count_call = pl.kernel(
    _count_body,
    out_type=(
        jax.ShapeDtypeStruct((NROWS, CH), _i32),
        jax.ShapeDtypeStruct((NROWS, CH), _i32),
        jax.ShapeDtypeStruct((RNP,), _f32),
    ),
    mesh=plsc.VectorSubcoreMesh(core_axis_name="c", subcore_axis_name="s"),
    scratch_types=[
        pltpu.VMEM((LCH, CH), _i32),
        pltpu.VMEM((LCH, CH), _i32),
        pltpu.VMEM((LCH, CH), _i32),
        pltpu.VMEM((LCH, CH), _i32),
        pltpu.VMEM((LCH, CH), _i32),
        pltpu.VMEM((CH,), _f32),
        pltpu.VMEM((SPT,), _f32),
        pltpu.VMEM_SHARED((RNP,), _f32),
    ],
)


def _scale_body(comb_hbm, inv_hbm, scale_hbm, combb, sbuf, sem):
    c = lax.axis_index("c")
    s = lax.axis_index("s")
    r0 = s * (NC * RPW) + jnp.where(c == 0, 0, A0)
    nrow = jnp.where(c == 0, A0, NC * RPW - A0)

    K = 8

    def burst(t, _):
        tb = t * K
        pltpu.sync_copy(comb_hbm.at[pl.ds(r0 + tb, K)], combb)
        for d in range(K):
            pltpu.async_copy(inv_hbm.at[combb.at[d]], sbuf.at[d], sem)
        for d in range(K):
            pltpu.make_async_copy(inv_hbm.at[combb.at[d]], sbuf.at[d],
                                  sem).wait()
        pltpu.sync_copy(sbuf, scale_hbm.at[pl.ds(r0 + tb, K)])
        return 0
    lax.fori_loop(0, nrow // K, burst, 0)


_scale_call = pl.kernel(
    _scale_body,
    out_type=jax.ShapeDtypeStruct((NROWS, CH), _f32),
    mesh=plsc.VectorSubcoreMesh(core_axis_name="c", subcore_axis_name="s"),
    scratch_types=[
        pltpu.VMEM((8, CH), _i32),
        pltpu.VMEM((8, CH), _f32),
        pltpu.SemaphoreType.DMA,
    ],
)


def _agg_body(table_hbm, idx_hbm, scale_hbm, dst_hbm, out_hbm,
              idxw, scalew, dstw, rb0, rb1, agg_sh, g0, g1):
    c = lax.axis_index("c")
    s = lax.axis_index("s")
    base = s * (NC * RPW) + jnp.where(c == 0, 0, A0)
    nwin = jnp.where(c == 0, A0 // 8, (NC * RPW - A0) // 8)
    rbufs = (rb0, rb1)
    gsems = (g0, g1)

    def zstep(i, _):
        for k in range(GP):
            rb0[i, pl.ds(k * 16, 16)] = jnp.zeros((16,), _f32)
        return 0
    lax.fori_loop(0, CH, zstep, 0)

    def zcopy(k, _):
        pltpu.sync_copy(rb0, agg_sh.at[pl.ds(s * NPT + k * ZR, ZR), :])
        return 0
    lax.fori_loop(0, NPT // ZR, zcopy, 0)
    plsc.subcore_barrier()

    def gather(j, b):
        pltpu.async_copy(table_hbm.at[idxw.at[j]], rbufs[b], gsems[b])

    def wait_g(b):
        pltpu.make_async_copy(table_hbm.at[idxw.at[0]], rbufs[b],
                              gsems[b]).wait()

    def compute(j, b):
        rb = rbufs[b]

        def grp(g, _):
            scv = scalew[j, pl.ds(g * 16, 16)]
            for l in range(16):
                sc = scv[l]
                i = g * 16 + l
                for k in range(H // 16):
                    rb[i, pl.ds(k * 16, 16)] = rb[i, pl.ds(k * 16, 16)] * sc
            return 0
        lax.fori_loop(0, GP, grp, 0)

    WIN = 8

    def mload(w):
        rbase = base + w * WIN
        pltpu.sync_copy(idx_hbm.at[pl.ds(rbase, WIN)], idxw)
        pltpu.sync_copy(scale_hbm.at[pl.ds(rbase, WIN)], scalew)
        pltpu.sync_copy(dst_hbm.at[pl.ds(rbase, WIN)], dstw)

    def wloop(w, _):
        mload(w)
        gather(0, 0)
        for j in range(WIN):
            b = j % 2
            if j + 1 < WIN:
                gather(j + 1, 1 - b)
            wait_g(b)
            compute(j, b)
            pltpu.sync_copy(rbufs[b], agg_sh.at[dstw.at[j]], add=True)
        return 0
    lax.fori_loop(0, nwin, wloop, 0)

    plsc.subcore_barrier()

    def wback(k, _):
        b0 = s * NPT + k * ZR
        pltpu.sync_copy(agg_sh.at[pl.ds(b0, ZR), :], rb0)
        pltpu.sync_copy(rb0, out_hbm.at[c, pl.ds(b0, ZR), :])
        return 0
    lax.fori_loop(0, NPT // ZR, wback, 0)


_agg_call = pl.kernel(
    _agg_body,
    out_type=jax.ShapeDtypeStruct((NC, NP, H), _f32),
    mesh=plsc.VectorSubcoreMesh(core_axis_name="c", subcore_axis_name="s"),
    scratch_types=[
        pltpu.VMEM((8, CH), _i32),
        pltpu.VMEM((8, CH), _f32),
        pltpu.VMEM((8, CH), _i32),
        pltpu.VMEM((CH, H), _f32),
        pltpu.VMEM((CH, H), _f32),
        pltpu.VMEM_SHARED((NP, H), _f32),
        pltpu.SemaphoreType.DMA,
        pltpu.SemaphoreType.DMA,
    ],
)


BN = 1000


def _xr_body(a0, a1, r1, b1, w2, x_out, xr_out):
    xb = a0[...] + a1[...] + r1[...] + b1[...][None, :]
    xb = jnp.where(xb > 0, xb, jnp.exp(xb) - 1.0)
    x_out[...] = xb
    for r in range(R):
        xr_out[r] = jnp.dot(xb, w2[r], preferred_element_type=_f32)


_xr_call = pl.pallas_call(
    _xr_body,
    grid=(N // BN,),
    in_specs=[
        pl.BlockSpec((BN, H), lambda i: (i, 0)),
        pl.BlockSpec((BN, H), lambda i: (i, 0)),
        pl.BlockSpec((BN, H), lambda i: (i, 0)),
        pl.BlockSpec((H,), lambda i: (0,)),
        pl.BlockSpec((R, H, H), lambda i: (0, 0, 0)),
    ],
    out_specs=[
        pl.BlockSpec((BN, H), lambda i: (i, 0)),
        pl.BlockSpec((R, BN, H), lambda i: (0, i, 0)),
    ],
    out_shape=[
        jax.ShapeDtypeStruct((N, H), _f32),
        jax.ShapeDtypeStruct((R, N, H), _f32),
    ],
)


def _fin_body(a0, a1, x, r2, b2, lw, lb, o):
    y = a0[...] + a1[...] + jnp.dot(x[...], r2[...], preferred_element_type=_f32)
    y = y + b2[...][None, :]
    y = jnp.where(y > 0, y, jnp.exp(y) - 1.0)
    o[...] = jnp.dot(y, lw[...], preferred_element_type=_f32) + lb[...][None, :]


_fin_call = pl.pallas_call(
    _fin_body,
    grid=(N // BN,),
    in_specs=[
        pl.BlockSpec((BN, H), lambda i: (i, 0)),
        pl.BlockSpec((BN, H), lambda i: (i, 0)),
        pl.BlockSpec((BN, H), lambda i: (i, 0)),
        pl.BlockSpec((H, H), lambda i: (0, 0)),
        pl.BlockSpec((H,), lambda i: (0,)),
        pl.BlockSpec((H, OUT), lambda i: (0, 0)),
        pl.BlockSpec((OUT,), lambda i: (0,)),
    ],
    out_specs=pl.BlockSpec((BN, OUT), lambda i: (i, 0)),
    out_shape=jax.ShapeDtypeStruct((N, OUT), _f32),
)


def kernel(edge_index, edge_type, weight1, root1, bias1, weight2, root2,
           bias2, lin_w, lin_b):
    src2 = jnp.concatenate(
        [edge_index[0], jnp.zeros((PAD,), _i32)]).reshape(NROWS, CH)
    dst2 = jnp.concatenate(
        [edge_index[1], jnp.full((PAD,), N, _i32)]).reshape(NROWS, CH)
    rel2 = jnp.concatenate(
        [edge_type, jnp.full((PAD,), R - 1, _i32)]).reshape(NROWS, CH)
    w1idx2, comb2, inv = _count_call(src2, dst2, rel2)
    scale2 = _scale_call(comb2, inv)

    agg1 = _agg_call(weight1.reshape(RN, H), w1idx2, scale2, dst2)
    x, xr = _xr_call(agg1[0, :N], agg1[1, :N], root1, bias1, weight2)

    agg2 = _agg_call(xr.reshape(RN, H), w1idx2, scale2, dst2)
    out = _fin_call(agg2[0, :N], agg2[1, :N], x, root2, bias2, lin_w, lin_b)
    return out
